# baseline (speedup 1.0000x reference)
import functools

import jax
import jax.numpy as jnp
import numpy as np
from jax import lax
from jax.experimental import pallas as pl
from jax.experimental.pallas import tpu as pltpu
from jax.experimental.pallas import tpu_sc as plsc

_MAPPING = np.array([0, 1, 1, 1, 1, 2, 2, 2, 2, 3, 3, 3, 3, 3, 4, 4, 4, 4, 5, 6, 7, 7, 7],
                    dtype=np.int32)
_DISTANCES = np.array([0, 0, 1, 2, 3, 0, 1, 2, 3, 0, 1, 2, 3, 4, 0, 1, 2, 3, 0, 0, 0, 1, 2],
                      dtype=np.float32)
_NCODE = 23
_NMAJOR = 8
_EMBED = 32
_TROWS = 32

_OH = np.zeros((_TROWS, _NMAJOR), np.float32)
_OH[np.arange(_NCODE), _MAPPING] = 1.0
_SOH = _OH * np.pad(_DISTANCES, (0, _TROWS - _NCODE))[:, None]

_B = 16 * 1 * 224 * 224
_NW = 32
_BPW = _B // _NW
_CH = 1568
_NCHUNK = _BPW // _CH
_UNROLL = 4


def _table_body(vec_ref, bias_ref, oh_ref, soh_ref, tab_ref):
    tab_ref[:, :] = (
        jnp.dot(oh_ref[:, :], bias_ref[:, :], preferred_element_type=jnp.float32,
                precision=jax.lax.Precision.HIGHEST)
        + jnp.dot(soh_ref[:, :], vec_ref[:, :], preferred_element_type=jnp.float32,
                  precision=jax.lax.Precision.HIGHEST)
    )


def _build_table(vectors, bias):
    return pl.pallas_call(
        _table_body,
        out_shape=jax.ShapeDtypeStruct((_TROWS, _EMBED), jnp.float32),
    )(vectors, bias, jnp.asarray(_OH), jnp.asarray(_SOH))


_mesh = plsc.VectorSubcoreMesh(core_axis_name="c", subcore_axis_name="s")


@functools.partial(
    pl.kernel,
    mesh=_mesh,
    out_type=jax.ShapeDtypeStruct((_B, _EMBED), jnp.float32),
    scratch_types=[
        pltpu.VMEM((2, _CH), jnp.int32),
        pltpu.VMEM((2, _CH, _EMBED), jnp.float32),
        pltpu.VMEM((_TROWS * _EMBED,), jnp.float32),
        pltpu.SemaphoreType.DMA,
        pltpu.SemaphoreType.DMA,
    ],
    compiler_params=pltpu.CompilerParams(use_tc_tiling_on_sc=False, needs_layout_passes=False),
)
def _gather_kernel(idx_hbm, tab_hbm, out_hbm, idx_v, rows_v, tab_v, ssem0, ssem1):
    cid = lax.axis_index("c")
    sid = lax.axis_index("s")
    wid = sid * 2 + cid
    base = wid * _BPW

    pltpu.sync_copy(tab_hbm, tab_v)

    lo = lax.iota(jnp.int32, 16)
    hi = lo + 16

    def fill_chunk(j, b):
        pltpu.sync_copy(idx_hbm.at[pl.ds(base + j * _CH, _CH)], idx_v.at[b])

        def body(k, carry):
            cvec32 = idx_v[b, pl.ds(k * 16, 16)] * _EMBED
            for l in range(16):
                cv = jnp.full((16,), cvec32[l], jnp.int32)
                i = k * 16 + l
                rows_v[b, i, pl.ds(0, 16)] = plsc.load_gather(tab_v, [cv + lo])
                rows_v[b, i, pl.ds(16, 16)] = plsc.load_gather(tab_v, [cv + hi])
            return carry

        lax.fori_loop(0, _CH // 16, body, 0)

    ssems = (ssem0, ssem1)
    sh = [None, None]
    for j in range(_NCHUNK):
        b = j % 2
        if sh[b] is not None:
            sh[b].wait()
        fill_chunk(j, b)
        sh[b] = pltpu.async_copy(
            rows_v.at[b], out_hbm.at[pl.ds(base + j * _CH, _CH)], ssems[b]
        )
    sh[0].wait()
    sh[1].wait()


def kernel(input, vectors, bias):
    table = _build_table(vectors, bias)
    idx = input.reshape(_B)
    out = _gather_kernel(idx, table.reshape(_TROWS * _EMBED))
    return out.reshape(input.shape + (_EMBED,))


# speedup vs baseline: 1.2146x; 1.2146x over previous
import functools

import jax
import jax.numpy as jnp
import numpy as np
from jax import lax
from jax.experimental import pallas as pl
from jax.experimental.pallas import tpu as pltpu
from jax.experimental.pallas import tpu_sc as plsc

_MAPPING = np.array([0, 1, 1, 1, 1, 2, 2, 2, 2, 3, 3, 3, 3, 3, 4, 4, 4, 4, 5, 6, 7, 7, 7],
                    dtype=np.int32)
_DISTANCES = np.array([0, 0, 1, 2, 3, 0, 1, 2, 3, 0, 1, 2, 3, 4, 0, 1, 2, 3, 0, 0, 0, 1, 2],
                      dtype=np.float32)
_NCODE = 23
_NMAJOR = 8
_EMBED = 32
_TROWS = 32

_OH = np.zeros((_TROWS, _NMAJOR), np.float32)
_OH[np.arange(_NCODE), _MAPPING] = 1.0
_SOH = _OH * np.pad(_DISTANCES, (0, _TROWS - _NCODE))[:, None]

_B = 16 * 1 * 224 * 224
_NW = 32
_BPW = _B // _NW
_CH = 1568
_NCHUNK = _BPW // _CH
_SP = 896
_NV = _CH - _SP


def _table_body(vec_ref, bias_ref, oh_ref, soh_ref, tab_ref):
    tab_ref[:, :] = (
        jnp.dot(oh_ref[:, :], bias_ref[:, :], preferred_element_type=jnp.float32,
                precision=jax.lax.Precision.HIGHEST)
        + jnp.dot(soh_ref[:, :], vec_ref[:, :], preferred_element_type=jnp.float32,
                  precision=jax.lax.Precision.HIGHEST)
    )


def _build_table(vectors, bias):
    return pl.pallas_call(
        _table_body,
        out_shape=jax.ShapeDtypeStruct((_TROWS, _EMBED), jnp.float32),
    )(vectors, bias, jnp.asarray(_OH), jnp.asarray(_SOH))


_mesh = plsc.VectorSubcoreMesh(core_axis_name="c", subcore_axis_name="s")


@functools.partial(
    pl.kernel,
    mesh=_mesh,
    out_type=jax.ShapeDtypeStruct((_B, _EMBED), jnp.float32),
    scratch_types=[
        pltpu.VMEM((2, _CH), jnp.int32),
        pltpu.VMEM((2, _CH, _EMBED), jnp.float32),
        pltpu.VMEM((_TROWS, _EMBED), jnp.float32),
        pltpu.VMEM_SHARED((_TROWS, _EMBED), jnp.float32),
        pltpu.SemaphoreType.DMA,
        pltpu.SemaphoreType.DMA,
        pltpu.SemaphoreType.DMA,
    ],
    compiler_params=pltpu.CompilerParams(use_tc_tiling_on_sc=False, needs_layout_passes=False),
)
def _gather_kernel(idx_hbm, tab_hbm, out_hbm, idx_v, rows_v, tab_v, tab_sh, gsem, ssem0, ssem1):
    cid = lax.axis_index("c")
    sid = lax.axis_index("s")
    wid = sid * 2 + cid
    base = wid * _BPW

    pltpu.sync_copy(tab_hbm, tab_v)

    @pl.when(sid == 0)
    def _():
        pltpu.sync_copy(tab_hbm, tab_sh)

    plsc.subcore_barrier()

    lo = lax.iota(jnp.int32, 16)
    hi = lo + 16

    def fill_chunk(j, b):
        pltpu.sync_copy(idx_hbm.at[pl.ds(base + j * _CH, _CH)], idx_v.at[b])

        gh = pltpu.async_copy(
            tab_sh.at[idx_v.at[b, pl.ds(0, _SP)]],
            rows_v.at[b, pl.ds(0, _SP)],
            gsem,
        )

        def body(k, carry):
            cvec = idx_v[b, pl.ds(_SP + k * 16, 16)]
            for l in range(16):
                cv = jnp.full((16,), cvec[l], jnp.int32)
                i = _SP + k * 16 + l
                rows_v[b, i, pl.ds(0, 16)] = plsc.load_gather(tab_v, [cv, lo])
                rows_v[b, i, pl.ds(16, 16)] = plsc.load_gather(tab_v, [cv, hi])
            return carry

        lax.fori_loop(0, _NV // 16, body, 0)
        gh.wait()

    ssems = (ssem0, ssem1)
    sh = [None, None]
    for j in range(_NCHUNK):
        b = j % 2
        if sh[b] is not None:
            sh[b].wait()
        fill_chunk(j, b)
        sh[b] = pltpu.async_copy(
            rows_v.at[b], out_hbm.at[pl.ds(base + j * _CH, _CH)], ssems[b]
        )
    sh[0].wait()
    sh[1].wait()


def kernel(input, vectors, bias):
    table = _build_table(vectors, bias)
    idx = input.reshape(_B)
    out = _gather_kernel(idx, table)
    return out.reshape(input.shape + (_EMBED,))


# speedup vs baseline: 1.3465x; 1.1086x over previous
import functools

import jax
import jax.numpy as jnp
import numpy as np
from jax import lax
from jax.experimental import pallas as pl
from jax.experimental.pallas import tpu as pltpu
from jax.experimental.pallas import tpu_sc as plsc

_MAPPING = np.array([0, 1, 1, 1, 1, 2, 2, 2, 2, 3, 3, 3, 3, 3, 4, 4, 4, 4, 5, 6, 7, 7, 7],
                    dtype=np.int32)
_DISTANCES = np.array([0, 0, 1, 2, 3, 0, 1, 2, 3, 0, 1, 2, 3, 4, 0, 1, 2, 3, 0, 0, 0, 1, 2],
                      dtype=np.float32)
_NCODE = 23
_NMAJOR = 8
_EMBED = 32
_TROWS = 32
_PROWS = 1024

_OH = np.zeros((_TROWS, _NMAJOR), np.float32)
_OH[np.arange(_NCODE), _MAPPING] = 1.0
_SOH = _OH * np.pad(_DISTANCES, (0, _TROWS - _NCODE))[:, None]
_PHI = np.zeros((_PROWS, _TROWS), np.float32)
_PHI[np.arange(_PROWS), np.arange(_PROWS) >> 5] = 1.0
_PLO = np.zeros((_PROWS, _TROWS), np.float32)
_PLO[np.arange(_PROWS), np.arange(_PROWS) & 31] = 1.0

_B = 16 * 1 * 224 * 224
_NP = _B // 2
_NW = 32
_PPW = _NP // _NW
_CHP = 784
_NCHUNK = _PPW // _CHP


def _table_body(vec_ref, bias_ref, oh_ref, soh_ref, phi_ref, plo_ref, tab2_ref):
    tab = (
        jnp.dot(oh_ref[:, :], bias_ref[:, :], preferred_element_type=jnp.float32,
                precision=jax.lax.Precision.HIGHEST)
        + jnp.dot(soh_ref[:, :], vec_ref[:, :], preferred_element_type=jnp.float32,
                  precision=jax.lax.Precision.HIGHEST)
    )
    tab2_ref[:, 0:_EMBED] = jnp.dot(
        phi_ref[:, :], tab, preferred_element_type=jnp.float32,
        precision=jax.lax.Precision.HIGHEST)
    tab2_ref[:, _EMBED:2 * _EMBED] = jnp.dot(
        plo_ref[:, :], tab, preferred_element_type=jnp.float32,
        precision=jax.lax.Precision.HIGHEST)


def _build_pair_table(vectors, bias):
    return pl.pallas_call(
        _table_body,
        out_shape=jax.ShapeDtypeStruct((_PROWS, 2 * _EMBED), jnp.float32),
    )(vectors, bias, jnp.asarray(_OH), jnp.asarray(_SOH),
      jnp.asarray(_PHI), jnp.asarray(_PLO))


_mesh = plsc.VectorSubcoreMesh(core_axis_name="c", subcore_axis_name="s")


@functools.partial(
    pl.kernel,
    mesh=_mesh,
    out_type=jax.ShapeDtypeStruct((_NP, 2 * _EMBED), jnp.float32),
    scratch_types=[
        pltpu.VMEM((2, 2 * _CHP), jnp.int32),
        pltpu.VMEM((2, _CHP), jnp.int32),
        pltpu.VMEM((2, _CHP, 2 * _EMBED), jnp.float32),
        pltpu.VMEM_SHARED((_PROWS, 2 * _EMBED), jnp.float32),
        pltpu.SemaphoreType.DMA,
        pltpu.SemaphoreType.DMA,
        pltpu.SemaphoreType.DMA,
    ],
    compiler_params=pltpu.CompilerParams(
        use_tc_tiling_on_sc=False, needs_layout_passes=False),
)
def _gather_kernel(idx_hbm, tab2_hbm, out_hbm, idx_v, pidx_v, rows_v, tab2_sh,
                   gsem, ssem0, ssem1):
    cid = lax.axis_index("c")
    sid = lax.axis_index("s")
    wid = sid * 2 + cid
    base = wid * _PPW
    pbase = base * 2

    @pl.when(sid == 0)
    def _():
        pltpu.sync_copy(tab2_hbm, tab2_sh)

    plsc.subcore_barrier()

    lo = lax.iota(jnp.int32, 16)
    ev = lo * 2
    od = ev + 1
    bz = jnp.zeros((16,), jnp.int32)
    bo = bz + 1

    def fill_chunk(j, b):
        pltpu.sync_copy(idx_hbm.at[pl.ds(pbase + j * 2 * _CHP, 2 * _CHP)],
                        idx_v.at[b])
        bv = bz if b == 0 else bo

        def body(k, carry):
            s = jnp.full((16,), k * 32, jnp.int32)
            c0 = plsc.load_gather(idx_v, [bv, s + ev])
            c1 = plsc.load_gather(idx_v, [bv, s + od])
            pidx_v[b, pl.ds(k * 16, 16)] = c0 * _TROWS + c1
            return carry

        lax.fori_loop(0, _CHP // 16, body, 0)

        pltpu.async_copy(
            tab2_sh.at[pidx_v.at[b]], rows_v.at[b], gsem
        ).wait()

    ssems = (ssem0, ssem1)
    sh = [None, None]
    for j in range(_NCHUNK):
        b = j % 2
        if sh[b] is not None:
            sh[b].wait()
        fill_chunk(j, b)
        sh[b] = pltpu.async_copy(
            rows_v.at[b], out_hbm.at[pl.ds(base + j * _CHP, _CHP)], ssems[b]
        )
    sh[0].wait()
    sh[1].wait()


def kernel(input, vectors, bias):
    table2 = _build_pair_table(vectors, bias)
    idx = input.reshape(_B)
    out = _gather_kernel(idx, table2)
    return out.reshape(input.shape + (_EMBED,))


# speedup vs baseline: 1.3475x; 1.0008x over previous
import functools

import jax
import jax.numpy as jnp
import numpy as np
from jax import lax
from jax.experimental import pallas as pl
from jax.experimental.pallas import tpu as pltpu
from jax.experimental.pallas import tpu_sc as plsc

_MAPPING = np.array([0, 1, 1, 1, 1, 2, 2, 2, 2, 3, 3, 3, 3, 3, 4, 4, 4, 4, 5, 6, 7, 7, 7],
                    dtype=np.int32)
_DISTANCES = np.array([0, 0, 1, 2, 3, 0, 1, 2, 3, 0, 1, 2, 3, 4, 0, 1, 2, 3, 0, 0, 0, 1, 2],
                      dtype=np.float32)
_NCODE = 23
_NMAJOR = 8
_EMBED = 32
_RSTRIDE = 33
_NREP = 16
_RROWS = _NREP * _RSTRIDE

_OH1 = np.zeros((_NCODE, _NMAJOR), np.float32)
_OH1[np.arange(_NCODE), _MAPPING] = 1.0
_SOH1 = _OH1 * _DISTANCES[:, None]
_OH = np.zeros((_RROWS, _NMAJOR), np.float32)
_SOH = np.zeros((_RROWS, _NMAJOR), np.float32)
for _r in range(_NREP):
    _OH[_r * _RSTRIDE:_r * _RSTRIDE + _NCODE] = _OH1
    _SOH[_r * _RSTRIDE:_r * _RSTRIDE + _NCODE] = _SOH1

_B = 16 * 1 * 224 * 224
_NW = 32
_BPW = _B // _NW
_CH = 1568
_NCHUNK = _BPW // _CH


def _table_body(vec_ref, bias_ref, oh_ref, soh_ref, tab_ref):
    tab_ref[:, :] = (
        jnp.dot(oh_ref[:, :], bias_ref[:, :], preferred_element_type=jnp.float32,
                precision=jax.lax.Precision.HIGHEST)
        + jnp.dot(soh_ref[:, :], vec_ref[:, :], preferred_element_type=jnp.float32,
                  precision=jax.lax.Precision.HIGHEST)
    )


def _build_table(vectors, bias):
    return pl.pallas_call(
        _table_body,
        out_shape=jax.ShapeDtypeStruct((_RROWS, _EMBED), jnp.float32),
    )(vectors, bias, jnp.asarray(_OH), jnp.asarray(_SOH))


_mesh = plsc.VectorSubcoreMesh(core_axis_name="c", subcore_axis_name="s")


@functools.partial(
    pl.kernel,
    mesh=_mesh,
    out_type=jax.ShapeDtypeStruct((_B, _EMBED), jnp.float32),
    scratch_types=[
        pltpu.VMEM((2, _CH), jnp.int32),
        pltpu.VMEM((2, _CH), jnp.int32),
        pltpu.VMEM((2, _CH, _EMBED), jnp.float32),
        pltpu.VMEM_SHARED((_RROWS, _EMBED), jnp.float32),
        pltpu.SemaphoreType.DMA,
        pltpu.SemaphoreType.DMA,
        pltpu.SemaphoreType.DMA,
    ],
    compiler_params=pltpu.CompilerParams(
        use_tc_tiling_on_sc=False, needs_layout_passes=False),
)
def _gather_kernel(idx_hbm, tab_hbm, out_hbm, idx_v, ridx_v, rows_v, tab_sh,
                   gsem, ssem0, ssem1):
    cid = lax.axis_index("c")
    sid = lax.axis_index("s")
    wid = sid * 2 + cid
    base = wid * _BPW

    @pl.when(sid == 0)
    def _():
        pltpu.sync_copy(tab_hbm, tab_sh)

    plsc.subcore_barrier()

    off = jnp.full((16,), sid * _RSTRIDE, jnp.int32)

    def fill_chunk(j, b):
        pltpu.sync_copy(idx_hbm.at[pl.ds(base + j * _CH, _CH)], idx_v.at[b])

        def body(k, carry):
            ridx_v[b, pl.ds(k * 16, 16)] = idx_v[b, pl.ds(k * 16, 16)] + off
            return carry

        lax.fori_loop(0, _CH // 16, body, 0)

        pltpu.async_copy(
            tab_sh.at[ridx_v.at[b]], rows_v.at[b], gsem
        ).wait()

    ssems = (ssem0, ssem1)
    sh = [None, None]
    for j in range(_NCHUNK):
        b = j % 2
        if sh[b] is not None:
            sh[b].wait()
        fill_chunk(j, b)
        sh[b] = pltpu.async_copy(
            rows_v.at[b], out_hbm.at[pl.ds(base + j * _CH, _CH)], ssems[b]
        )
    sh[0].wait()
    sh[1].wait()


def kernel(input, vectors, bias):
    table = _build_table(vectors, bias)
    idx = input.reshape(_B)
    out = _gather_kernel(idx, table)
    return out.reshape(input.shape + (_EMBED,))


# speedup vs baseline: 1.3713x; 1.0177x over previous
import functools

import jax
import jax.numpy as jnp
import numpy as np
from jax import lax
from jax.experimental import pallas as pl
from jax.experimental.pallas import tpu as pltpu
from jax.experimental.pallas import tpu_sc as plsc

_MAPPING = np.array([0, 1, 1, 1, 1, 2, 2, 2, 2, 3, 3, 3, 3, 3, 4, 4, 4, 4, 5, 6, 7, 7, 7],
                    dtype=np.int32)
_DISTANCES = np.array([0, 0, 1, 2, 3, 0, 1, 2, 3, 0, 1, 2, 3, 4, 0, 1, 2, 3, 0, 0, 0, 1, 2],
                      dtype=np.float32)
_NCODE = 23
_NMAJOR = 8
_EMBED = 32
_TROWS = 32

_OH = np.zeros((_TROWS, _NMAJOR), np.float32)
_OH[np.arange(_NCODE), _MAPPING] = 1.0
_SOH = _OH * np.pad(_DISTANCES, (0, _TROWS - _NCODE))[:, None]

_B = 16 * 1 * 224 * 224
_NW = 32
_BPW = _B // _NW
_CH = 1568
_NCHUNK = _BPW // _CH


def _table_body(vec_ref, bias_ref, oh_ref, soh_ref, tab_ref):
    tab_ref[:, :] = (
        jnp.dot(oh_ref[:, :], bias_ref[:, :], preferred_element_type=jnp.float32,
                precision=jax.lax.Precision.HIGHEST)
        + jnp.dot(soh_ref[:, :], vec_ref[:, :], preferred_element_type=jnp.float32,
                  precision=jax.lax.Precision.HIGHEST)
    )


def _build_table(vectors, bias):
    return pl.pallas_call(
        _table_body,
        out_shape=jax.ShapeDtypeStruct((_TROWS, _EMBED), jnp.float32),
    )(vectors, bias, jnp.asarray(_OH), jnp.asarray(_SOH))


_mesh = plsc.VectorSubcoreMesh(core_axis_name="c", subcore_axis_name="s")


@functools.partial(
    pl.kernel,
    mesh=_mesh,
    out_type=jax.ShapeDtypeStruct((_B, _EMBED), jnp.float32),
    scratch_types=[
        pltpu.VMEM((_BPW,), jnp.int32),
        pltpu.VMEM((2, _CH, _EMBED), jnp.float32),
        pltpu.VMEM_SHARED((_TROWS, _EMBED), jnp.float32),
        pltpu.SemaphoreType.DMA,
        pltpu.SemaphoreType.DMA,
        pltpu.SemaphoreType.DMA,
    ],
    compiler_params=pltpu.CompilerParams(use_tc_tiling_on_sc=False),
)
def _gather_kernel(idx_hbm, tab_hbm, out_hbm, idx_v, rows_v, tab_sh, gsem, ssem0, ssem1):
    cid = lax.axis_index("c")
    sid = lax.axis_index("s")
    wid = sid * 2 + cid
    base = wid * _BPW

    @pl.when(sid == 0)
    def _():
        pltpu.sync_copy(tab_hbm, tab_sh)

    plsc.subcore_barrier()

    pltpu.sync_copy(idx_hbm.at[pl.ds(base, _BPW)], idx_v)

    ssems = (ssem0, ssem1)
    handles = [None, None]
    for j in range(_NCHUNK):
        b = j % 2
        if handles[b] is not None:
            handles[b].wait()
        pltpu.async_copy(
            tab_sh.at[idx_v.at[pl.ds(j * _CH, _CH)]], rows_v.at[b], gsem
        ).wait()
        handles[b] = pltpu.async_copy(
            rows_v.at[b], out_hbm.at[pl.ds(base + j * _CH, _CH)], ssems[b]
        )
    handles[0].wait()
    handles[1].wait()


def kernel(input, vectors, bias):
    table = _build_table(vectors, bias)
    idx = input.reshape(_B)
    out = _gather_kernel(idx, table)
    return out.reshape(input.shape + (_EMBED,))
